# baseline (speedup 1.0000x reference)
import functools

import jax
import jax.numpy as jnp
from jax import lax
from jax.experimental import pallas as pl
from jax.experimental.pallas import tpu as pltpu
from jax.experimental.pallas import tpu_sc as plsc

BATCH = 16384
NUM_FIELDS = 26
VOCAB = 1000000

NW = 32
CB = BATCH // NW
CHUNK = 128
NCH = NUM_FIELDS * CB // CHUNK
CPF = CB // CHUNK
GRP = 8
NGRP = NCH // GRP

_mesh = plsc.VectorSubcoreMesh(core_axis_name="c", subcore_axis_name="s")


@functools.partial(
    pl.kernel,
    mesh=_mesh,
    out_type=jax.ShapeDtypeStruct((BATCH,), jnp.float32),
    scratch_types=[
        pltpu.VMEM((NCH, CHUNK), jnp.int32),
        pltpu.VMEM((NCH, CHUNK), jnp.float32),
        pltpu.VMEM((CB,), jnp.float32),
        pltpu.VMEM((16,), jnp.float32),
        pltpu.SemaphoreType.DMA,
    ],
)
def _gather_sum(idx_hbm, w_hbm, bias_hbm, out_hbm, idx_v, vals_v, acc_v,
                bias_v, sem):
    wid = lax.axis_index("s") * 2 + lax.axis_index("c")

    pltpu.sync_copy(idx_hbm.at[wid], idx_v)
    pltpu.sync_copy(bias_hbm, bias_v)

    def off_body(r, carry):
        off = (r // CPF) * VOCAB
        for s in range(CHUNK // 16):
            sl = pl.ds(s * 16, 16)
            idx_v[r, sl] = idx_v[r, sl] + off
        return carry

    lax.fori_loop(0, NCH, off_body, 0)

    def fire_group(g):
        for k in range(GRP):
            r = g * GRP + k
            pltpu.async_copy(w_hbm.at[idx_v.at[r]], vals_v.at[r], sem)

    def drain_one(r):
        pltpu.make_async_copy(w_hbm.at[pl.ds(0, CHUNK)], vals_v.at[r],
                              sem).wait()

    fire_group(0)

    def pipe_body(g, carry):
        fire_group(g + 1)
        for k in range(GRP):
            drain_one(g * GRP + k)
        return carry

    lax.fori_loop(0, NGRP - 1, pipe_body, 0)
    for k in range(GRP):
        drain_one((NGRP - 1) * GRP + k)

    def acc_body(c, carry):
        base_r = c // (CHUNK // 16)
        off = (c % (CHUNK // 16)) * 16
        v = bias_v[...]
        for f in range(NUM_FIELDS):
            v = v + vals_v[CPF * f + base_r, pl.ds(off, 16)]
        acc_v[pl.ds(c * 16, 16)] = v
        return carry

    lax.fori_loop(0, CB // 16, acc_body, 0)

    pltpu.sync_copy(acc_v, out_hbm.at[pl.ds(wid * CB, CB)])


def kernel(indices, W, bias):
    idx32 = indices.astype(jnp.int32)
    idx_prep = (idx32.T.reshape(NUM_FIELDS, NW, CB)
                .swapaxes(0, 1)
                .reshape(NW, NCH, CHUNK))
    w_flat = W.reshape(NUM_FIELDS * VOCAB)
    bias16 = jnp.broadcast_to(bias.astype(jnp.float32), (16,))
    out = _gather_sum(idx_prep, w_flat, bias16)
    return out.reshape(BATCH, 1)


# speedup vs baseline: 50.0966x; 50.0966x over previous
import functools

import jax
import jax.numpy as jnp
from jax import lax
from jax.experimental import pallas as pl
from jax.experimental.pallas import tpu as pltpu
from jax.experimental.pallas import tpu_sc as plsc

BATCH = 16384
NUM_FIELDS = 26
VOCAB = 1000000

RS = VOCAB + 64
WFLAT = NUM_FIELDS * RS

NW = 32
CB = BATCH // NW
CHUNK = 128
NCH = NUM_FIELDS * CB // CHUNK
CPF = CB // CHUNK
GRP = 8
NGRP = NCH // GRP

_mesh = plsc.VectorSubcoreMesh(core_axis_name="c", subcore_axis_name="s")


@functools.partial(
    pl.kernel,
    mesh=_mesh,
    out_type=jax.ShapeDtypeStruct((BATCH,), jnp.float32),

    scratch_types=[
        pltpu.VMEM((NCH, CHUNK), jnp.int32),
        pltpu.VMEM((NCH, CHUNK), jnp.float32),
        pltpu.VMEM((CB,), jnp.float32),
        pltpu.VMEM((16,), jnp.float32),
        pltpu.SemaphoreType.DMA,
    ],
)
def _gather_sum(idx_hbm, w_hbm, bias_hbm, out_hbm, idx_v, vals_v, acc_v,
                bias_v, sem):
    wid = lax.axis_index("s") * 2 + lax.axis_index("c")

    pltpu.sync_copy(idx_hbm.at[wid], idx_v)
    pltpu.sync_copy(bias_hbm, bias_v)

    def fire_group(g):
        for k in range(GRP):
            r = g * GRP + k
            f = r // CPF
            pltpu.async_copy(w_hbm.at[f, 0].at[idx_v.at[r]],
                             vals_v.at[r], sem)

    def drain_one(r):
        pltpu.make_async_copy(w_hbm.at[0, 0].at[pl.ds(0, CHUNK)],
                              vals_v.at[r], sem).wait()

    fire_group(0)

    def pipe_body(g, carry):
        fire_group(g + 1)
        for k in range(GRP):
            drain_one(g * GRP + k)
        return carry

    lax.fori_loop(0, NGRP - 1, pipe_body, 0)
    for k in range(GRP):
        drain_one((NGRP - 1) * GRP + k)

    def acc_body(c, carry):
        base_r = c // (CHUNK // 16)
        off = (c % (CHUNK // 16)) * 16
        v = bias_v[...]
        for f in range(NUM_FIELDS):
            v = v + vals_v[CPF * f + base_r, pl.ds(off, 16)]
        acc_v[pl.ds(c * 16, 16)] = v
        return carry

    lax.fori_loop(0, CB // 16, acc_body, 0)

    pltpu.sync_copy(acc_v, out_hbm.at[pl.ds(wid * CB, CB)])


def kernel(indices, W, bias):
    idx32 = indices.astype(jnp.int32)
    idx_prep = (idx32.T.reshape(NUM_FIELDS, NW, CB)
                .swapaxes(0, 1)
                .reshape(NW, NCH, CHUNK))
    w_nat = jnp.transpose(W, (0, 2, 1))
    bias16 = jnp.broadcast_to(bias.astype(jnp.float32), (16,))
    out = _gather_sum(idx_prep, w_nat, bias16)
    return out.reshape(BATCH, 1)
